# baseline (device time: 162382 ns/iter reference)
import jax
import jax.numpy as jnp
from jax import lax
from jax.experimental import pallas as pl
from jax.experimental.pallas import tpu as pltpu

N_DEV = 4
B, S_SHARD, H, D = 2, 512, 8, 64
BH = B * H
SCALE = D ** -0.5


def _attn_body(q_ref, k_ref, v_ref, out_ref, kv_ref, acc_ref, den_ref,
               send_sems, recv_sems):
    my = lax.axis_index("i")
    left = lax.rem(my + N_DEV - 1, N_DEV)
    right = lax.rem(my + 1, N_DEV)

    acc_ref[...] = jnp.zeros_like(acc_ref)
    den_ref[...] = jnp.zeros_like(den_ref)

    kv_ref[0, 0] = k_ref[...]
    kv_ref[0, 1] = v_ref[...]

    barrier_sem = pltpu.get_barrier_semaphore()
    for nbr in (left, right):
        pl.semaphore_signal(barrier_sem, inc=1, device_id=(nbr,),
                            device_id_type=pl.DeviceIdType.MESH)
    pl.semaphore_wait(barrier_sem, 2)

    def accum(slot):
        def head(i, _):
            q = q_ref[i]
            k = kv_ref[slot, 0, i]
            v = kv_ref[slot, 1, i]
            s = lax.dot_general(q, k, (((1,), (1,)), ((), ())),
                                preferred_element_type=jnp.float32)
            p = jnp.exp(s)
            den_ref[i] += jnp.sum(p, axis=1, keepdims=True)
            acc_ref[i] += lax.dot_general(p.astype(jnp.bfloat16), v,
                                          (((1,), (0,)), ((), ())),
                                          preferred_element_type=jnp.float32)
            return 0
        lax.fori_loop(0, BH, head, 0)

    for h in range(N_DEV - 1):
        rdma = pltpu.make_async_remote_copy(
            src_ref=kv_ref.at[h],
            dst_ref=kv_ref.at[h + 1],
            send_sem=send_sems.at[h],
            recv_sem=recv_sems.at[h],
            device_id=(right,),
            device_id_type=pl.DeviceIdType.MESH,
        )
        rdma.start()
        accum(h)
        rdma.wait()
    accum(N_DEV - 1)

    out_ref[...] = acc_ref[...] / den_ref[...]


def kernel(Q, K, V):
    qt = (jnp.transpose(Q, (0, 2, 1, 3)) * SCALE).astype(jnp.bfloat16)
    qt = qt.reshape(BH, S_SHARD, D)
    kt = jnp.transpose(K, (0, 2, 1, 3)).astype(jnp.bfloat16).reshape(BH, S_SHARD, D)
    vt = jnp.transpose(V, (0, 2, 1, 3)).astype(jnp.bfloat16).reshape(BH, S_SHARD, D)

    out = pl.pallas_call(
        _attn_body,
        out_shape=jax.ShapeDtypeStruct((BH, S_SHARD, D), jnp.float32),
        in_specs=[pl.BlockSpec(memory_space=pltpu.VMEM)] * 3,
        out_specs=pl.BlockSpec(memory_space=pltpu.VMEM),
        scratch_shapes=[
            pltpu.VMEM((N_DEV, 2, BH, S_SHARD, D), jnp.bfloat16),
            pltpu.VMEM((BH, S_SHARD, D), jnp.float32),
            pltpu.VMEM((BH, S_SHARD, 1), jnp.float32),
            pltpu.SemaphoreType.DMA((N_DEV - 1,)),
            pltpu.SemaphoreType.DMA((N_DEV - 1,)),
        ],
        compiler_params=pltpu.CompilerParams(collective_id=0),
    )(qt, kt, vt)

    return jnp.transpose(out.reshape(B, H, S_SHARD, D), (0, 2, 1, 3))


# device time: 92842 ns/iter; 1.7490x vs baseline; 1.7490x over previous
import jax
import jax.numpy as jnp
from jax import lax
from jax.experimental import pallas as pl
from jax.experimental.pallas import tpu as pltpu

N_DEV = 4
B, S_SHARD, H, D = 2, 512, 8, 64
BH = B * H
SCALE = D ** -0.5


def _attn_body(q_ref, k_ref, v_ref, out_ref, kv_ref, acc_ref, den_ref,
               send_sems, recv_sems):
    my = lax.axis_index("i")
    left = lax.rem(my + N_DEV - 1, N_DEV)
    right = lax.rem(my + 1, N_DEV)

    acc_ref[...] = jnp.zeros_like(acc_ref)
    den_ref[...] = jnp.zeros_like(den_ref)

    kv_ref[0, 0] = k_ref[...]
    kv_ref[0, 1] = v_ref[...]

    barrier_sem = pltpu.get_barrier_semaphore()
    for nbr in (left, right):
        pl.semaphore_signal(barrier_sem, inc=1, device_id=(nbr,),
                            device_id_type=pl.DeviceIdType.MESH)
    pl.semaphore_wait(barrier_sem, 2)

    def accum(slot):
        def head(i, _):
            q = q_ref[i]
            k = kv_ref[slot, 0, i]
            v = kv_ref[slot, 1, i]
            s = lax.dot_general(q, k, (((1,), (1,)), ((), ())),
                                preferred_element_type=jnp.float32)
            p = jnp.exp(s)
            den_ref[i] += jnp.sum(p, axis=1, keepdims=True)
            acc_ref[i] += lax.dot_general(p.astype(jnp.bfloat16), v,
                                          (((1,), (0,)), ((), ())),
                                          preferred_element_type=jnp.float32)
            return 0
        lax.fori_loop(0, BH, head, 0)

    def mk(src, dst, sem_idx, dev):
        return pltpu.make_async_remote_copy(
            src_ref=src, dst_ref=dst,
            send_sem=send_sems.at[sem_idx], recv_sem=recv_sems.at[sem_idx],
            device_id=(dev,), device_id_type=pl.DeviceIdType.MESH,
        )

    r1_right = mk(kv_ref.at[0], kv_ref.at[1], 0, right)
    r1_left = mk(kv_ref.at[0], kv_ref.at[2], 1, left)
    r2_right = mk(kv_ref.at[1, 0], kv_ref.at[3, 0], 2, right)
    r2_left = mk(kv_ref.at[2, 1], kv_ref.at[3, 1], 3, left)

    r1_right.start()
    r1_left.start()
    accum(0)

    r1_right.wait_recv()
    r2_right.start()
    r1_left.wait_recv()
    r2_left.start()

    accum(1)
    accum(2)

    r2_right.wait_recv()
    r2_left.wait_recv()
    accum(3)

    for r in (r1_right, r1_left, r2_right, r2_left):
        r.wait_send()

    out_ref[...] = acc_ref[...] / den_ref[...]


def kernel(Q, K, V):
    qt = (jnp.transpose(Q, (0, 2, 1, 3)) * SCALE).astype(jnp.bfloat16)
    qt = qt.reshape(BH, S_SHARD, D)
    kt = jnp.transpose(K, (0, 2, 1, 3)).astype(jnp.bfloat16).reshape(BH, S_SHARD, D)
    vt = jnp.transpose(V, (0, 2, 1, 3)).astype(jnp.bfloat16).reshape(BH, S_SHARD, D)

    out = pl.pallas_call(
        _attn_body,
        out_shape=jax.ShapeDtypeStruct((BH, S_SHARD, D), jnp.float32),
        in_specs=[pl.BlockSpec(memory_space=pltpu.VMEM)] * 3,
        out_specs=pl.BlockSpec(memory_space=pltpu.VMEM),
        scratch_shapes=[
            pltpu.VMEM((N_DEV, 2, BH, S_SHARD, D), jnp.bfloat16),
            pltpu.VMEM((BH, S_SHARD, D), jnp.float32),
            pltpu.VMEM((BH, S_SHARD, 1), jnp.float32),
            pltpu.SemaphoreType.DMA((4,)),
            pltpu.SemaphoreType.DMA((4,)),
        ],
        compiler_params=pltpu.CompilerParams(collective_id=0),
    )(qt, kt, vt)

    return jnp.transpose(out.reshape(B, H, S_SHARD, D), (0, 2, 1, 3))


# device time: 60885 ns/iter; 2.6670x vs baseline; 1.5249x over previous
import jax
import jax.numpy as jnp
from jax import lax
from jax.experimental import pallas as pl
from jax.experimental.pallas import tpu as pltpu

N_DEV = 4
B, S_SHARD, H, D = 2, 512, 8, 64
BH = B * H
HG = 4
SCALE = D ** -0.5


def _attn_body(q_ref, kq_ref, vq_ref, ksc_ref, vsc_ref, out_ref,
               kv_ref, sc_ref, acc_ref, den_ref, send_sems, recv_sems):
    my = lax.axis_index("i")
    left = lax.rem(my + N_DEV - 1, N_DEV)
    right = lax.rem(my + 1, N_DEV)

    acc_ref[...] = jnp.zeros_like(acc_ref)
    den_ref[...] = jnp.zeros_like(den_ref)

    kv_ref[0, 0] = kq_ref[...]
    kv_ref[0, 1] = vq_ref[...]
    sc_ref[0, 0] = ksc_ref[...]
    sc_ref[0, 1] = vsc_ref[...]

    barrier_sem = pltpu.get_barrier_semaphore()
    for nbr in (left, right):
        pl.semaphore_signal(barrier_sem, inc=1, device_id=(nbr,),
                            device_id_type=pl.DeviceIdType.MESH)
    pl.semaphore_wait(barrier_sem, 2)

    def accum(slot):
        for g in range(BH // HG):
            lo, hi = g * HG, (g + 1) * HG
            q = q_ref[lo:hi]
            k = kv_ref[slot, 0, lo:hi].astype(jnp.bfloat16)
            v = kv_ref[slot, 1, lo:hi].astype(jnp.bfloat16)
            s = lax.dot_general(q, k, (((2,), (2,)), ((0,), (0,))),
                                preferred_element_type=jnp.float32)
            s = s * ksc_broadcast(slot, 0, lo, hi)
            p = jnp.exp(s)
            den_ref[lo:hi] += jnp.sum(p, axis=2, keepdims=True)
            p = (p * ksc_broadcast(slot, 1, lo, hi)).astype(jnp.bfloat16)
            acc_ref[lo:hi] += lax.dot_general(p, v,
                                              (((2,), (1,)), ((0,), (0,))),
                                              preferred_element_type=jnp.float32)

    def ksc_broadcast(slot, kv, lo, hi):
        return sc_ref[slot, kv, lo:hi]

    def mk(src, dst, sem_idx, dev):
        return pltpu.make_async_remote_copy(
            src_ref=src, dst_ref=dst,
            send_sem=send_sems.at[sem_idx], recv_sem=recv_sems.at[sem_idx],
            device_id=(dev,), device_id_type=pl.DeviceIdType.MESH,
        )

    r1_right = mk(kv_ref.at[0], kv_ref.at[1], 0, right)
    r1_left = mk(kv_ref.at[0], kv_ref.at[2], 1, left)
    r2_right = mk(kv_ref.at[1, 0], kv_ref.at[3, 0], 2, right)
    r2_left = mk(kv_ref.at[2, 1], kv_ref.at[3, 1], 3, left)
    s1_right = mk(sc_ref.at[0], sc_ref.at[1], 4, right)
    s1_left = mk(sc_ref.at[0], sc_ref.at[2], 5, left)
    s2_right = mk(sc_ref.at[1, 0], sc_ref.at[3, 0], 6, right)
    s2_left = mk(sc_ref.at[2, 1], sc_ref.at[3, 1], 7, left)

    s1_right.start()
    s1_left.start()
    r1_right.start()
    r1_left.start()
    accum(0)

    s1_right.wait_recv()
    r1_right.wait_recv()
    s2_right.start()
    r2_right.start()
    s1_left.wait_recv()
    r1_left.wait_recv()
    s2_left.start()
    r2_left.start()

    accum(1)
    accum(2)

    s2_right.wait_recv()
    r2_right.wait_recv()
    s2_left.wait_recv()
    r2_left.wait_recv()
    accum(3)

    for r in (r1_right, r1_left, r2_right, r2_left,
              s1_right, s1_left, s2_right, s2_left):
        r.wait_send()

    out_ref[...] = acc_ref[...] / den_ref[...]


def kernel(Q, K, V):
    qt = (jnp.transpose(Q, (0, 2, 1, 3)) * SCALE).astype(jnp.bfloat16)
    qt = qt.reshape(BH, S_SHARD, D)
    kt = jnp.transpose(K, (0, 2, 1, 3)).reshape(BH, S_SHARD, D)
    vt = jnp.transpose(V, (0, 2, 1, 3)).reshape(BH, S_SHARD, D)

    def quant(x):
        m = jnp.max(jnp.abs(x), axis=2, keepdims=True)
        xq = jnp.round(x * (127.0 / m)).astype(jnp.int8)
        sc = jnp.transpose(m / 127.0, (0, 2, 1))
        return xq, sc

    kq, ksc = quant(kt)
    vq, vsc = quant(vt)

    out = pl.pallas_call(
        _attn_body,
        out_shape=jax.ShapeDtypeStruct((BH, S_SHARD, D), jnp.float32),
        in_specs=[pl.BlockSpec(memory_space=pltpu.VMEM)] * 5,
        out_specs=pl.BlockSpec(memory_space=pltpu.VMEM),
        scratch_shapes=[
            pltpu.VMEM((N_DEV, 2, BH, S_SHARD, D), jnp.int8),
            pltpu.VMEM((N_DEV, 2, BH, 1, S_SHARD), jnp.float32),
            pltpu.VMEM((BH, S_SHARD, D), jnp.float32),
            pltpu.VMEM((BH, S_SHARD, 1), jnp.float32),
            pltpu.SemaphoreType.DMA((8,)),
            pltpu.SemaphoreType.DMA((8,)),
        ],
        compiler_params=pltpu.CompilerParams(collective_id=0),
    )(qt, kq, vq, ksc, vsc)

    return jnp.transpose(out.reshape(B, H, S_SHARD, D), (0, 2, 1, 3))


# device time: 60497 ns/iter; 2.6841x vs baseline; 1.0064x over previous
import jax
import jax.numpy as jnp
from jax import lax
from jax.experimental import pallas as pl
from jax.experimental.pallas import tpu as pltpu

N_DEV = 4
B, S_SHARD, H, D = 2, 512, 8, 64
BH = B * H
HG = 4
SCALE = D ** -0.5


def _attn_body(q_ref, kq_ref, vq_ref, ksc_ref, vsc_ref, out_ref,
               kv_ref, sc_ref, acc_ref, den_ref, send_sems, recv_sems):
    my = lax.axis_index("i")
    left = lax.rem(my + N_DEV - 1, N_DEV)
    right = lax.rem(my + 1, N_DEV)

    acc_ref[...] = jnp.zeros_like(acc_ref)
    den_ref[...] = jnp.zeros_like(den_ref)

    kv_ref[0, 0] = kq_ref[...]
    kv_ref[0, 1] = vq_ref[...]
    sc_ref[0, 0] = ksc_ref[...]
    sc_ref[0, 1] = vsc_ref[...]

    barrier_sem = pltpu.get_barrier_semaphore()
    for nbr in (left, right):
        pl.semaphore_signal(barrier_sem, inc=1, device_id=(nbr,),
                            device_id_type=pl.DeviceIdType.MESH)
    pl.semaphore_wait(barrier_sem, 2)

    def accum(slot):
        for g in range(BH // HG):
            lo, hi = g * HG, (g + 1) * HG
            q = q_ref[lo:hi]
            k = kv_ref[slot, 0, lo:hi].astype(jnp.bfloat16)
            v = kv_ref[slot, 1, lo:hi].astype(jnp.bfloat16)
            s = lax.dot_general(q, k, (((2,), (2,)), ((0,), (0,))),
                                preferred_element_type=jnp.float32)
            s = s * ksc_broadcast(slot, 0, lo, hi)
            p = jnp.exp(s.astype(jnp.bfloat16))
            den_ref[lo:hi] += jnp.sum(p, axis=2, keepdims=True,
                                      dtype=jnp.float32)
            p = p * ksc_broadcast(slot, 1, lo, hi).astype(jnp.bfloat16)
            acc_ref[lo:hi] += lax.dot_general(p, v,
                                              (((2,), (1,)), ((0,), (0,))),
                                              preferred_element_type=jnp.float32)

    def ksc_broadcast(slot, kv, lo, hi):
        return sc_ref[slot, kv, lo:hi]

    def mk(src, dst, sem_idx, dev):
        return pltpu.make_async_remote_copy(
            src_ref=src, dst_ref=dst,
            send_sem=send_sems.at[sem_idx], recv_sem=recv_sems.at[sem_idx],
            device_id=(dev,), device_id_type=pl.DeviceIdType.MESH,
        )

    r1_right = mk(kv_ref.at[0], kv_ref.at[1], 0, right)
    r1_left = mk(kv_ref.at[0], kv_ref.at[2], 1, left)
    r2_right = mk(kv_ref.at[1, 0], kv_ref.at[3, 0], 2, right)
    r2_left = mk(kv_ref.at[2, 1], kv_ref.at[3, 1], 3, left)
    s1_right = mk(sc_ref.at[0], sc_ref.at[1], 4, right)
    s1_left = mk(sc_ref.at[0], sc_ref.at[2], 5, left)
    s2_right = mk(sc_ref.at[1, 0], sc_ref.at[3, 0], 6, right)
    s2_left = mk(sc_ref.at[2, 1], sc_ref.at[3, 1], 7, left)

    s1_right.start()
    s1_left.start()
    r1_right.start()
    r1_left.start()
    accum(0)

    s1_right.wait_recv()
    r1_right.wait_recv()
    s2_right.start()
    r2_right.start()
    s1_left.wait_recv()
    r1_left.wait_recv()
    s2_left.start()
    r2_left.start()

    accum(1)
    accum(2)

    s2_right.wait_recv()
    r2_right.wait_recv()
    s2_left.wait_recv()
    r2_left.wait_recv()
    accum(3)

    for r in (r1_right, r1_left, r2_right, r2_left,
              s1_right, s1_left, s2_right, s2_left):
        r.wait_send()

    out_ref[...] = acc_ref[...] / den_ref[...]


def kernel(Q, K, V):
    qt = (jnp.transpose(Q, (0, 2, 1, 3)) * SCALE).astype(jnp.bfloat16)
    qt = qt.reshape(BH, S_SHARD, D)
    kt = jnp.transpose(K, (0, 2, 1, 3)).reshape(BH, S_SHARD, D)
    vt = jnp.transpose(V, (0, 2, 1, 3)).reshape(BH, S_SHARD, D)

    def quant(x):
        m = jnp.max(jnp.abs(x), axis=2, keepdims=True)
        xq = jnp.round(x * (127.0 / m)).astype(jnp.int8)
        sc = jnp.transpose(m / 127.0, (0, 2, 1))
        return xq, sc

    kq, ksc = quant(kt)
    vq, vsc = quant(vt)

    out = pl.pallas_call(
        _attn_body,
        out_shape=jax.ShapeDtypeStruct((BH, S_SHARD, D), jnp.float32),
        in_specs=[pl.BlockSpec(memory_space=pltpu.VMEM)] * 5,
        out_specs=pl.BlockSpec(memory_space=pltpu.VMEM),
        scratch_shapes=[
            pltpu.VMEM((N_DEV, 2, BH, S_SHARD, D), jnp.int8),
            pltpu.VMEM((N_DEV, 2, BH, 1, S_SHARD), jnp.float32),
            pltpu.VMEM((BH, S_SHARD, D), jnp.float32),
            pltpu.VMEM((BH, S_SHARD, 1), jnp.float32),
            pltpu.SemaphoreType.DMA((8,)),
            pltpu.SemaphoreType.DMA((8,)),
        ],
        compiler_params=pltpu.CompilerParams(collective_id=0),
    )(qt, kq, vq, ksc, vsc)

    return jnp.transpose(out.reshape(B, H, S_SHARD, D), (0, 2, 1, 3))


# device time: 56937 ns/iter; 2.8520x vs baseline; 1.0625x over previous
import jax
import jax.numpy as jnp
from jax import lax
from jax.experimental import pallas as pl
from jax.experimental.pallas import tpu as pltpu

N_DEV = 4
B, S_SHARD, H, D = 2, 512, 8, 64
BH = B * H
HG = 4
HH = BH // 2
SCALE = D ** -0.5


def _attn_body(q_ref, kq_ref, vq_ref, ksc_ref, vsc_ref, out_ref,
               kv_ref, sc_ref, acc_ref, den_ref, send_sems, recv_sems):
    my = lax.axis_index("i")
    left = lax.rem(my + N_DEV - 1, N_DEV)
    right = lax.rem(my + 1, N_DEV)

    acc_ref[...] = jnp.zeros_like(acc_ref)
    den_ref[...] = jnp.zeros_like(den_ref)

    kv_ref[0, 0, 0] = kq_ref[:HH]
    kv_ref[0, 1, 0] = kq_ref[HH:]
    kv_ref[0, 0, 1] = vq_ref[:HH]
    kv_ref[0, 1, 1] = vq_ref[HH:]
    sc_ref[0, 0] = ksc_ref[...]
    sc_ref[0, 1] = vsc_ref[...]

    barrier_sem = pltpu.get_barrier_semaphore()
    for nbr in (left, right):
        pl.semaphore_signal(barrier_sem, inc=1, device_id=(nbr,),
                            device_id_type=pl.DeviceIdType.MESH)
    pl.semaphore_wait(barrier_sem, 2)

    def accum(slot, g):
        hf, l4 = divmod(g, 2)
        lo4 = l4 * HG
        lo, hi = g * HG, (g + 1) * HG
        q = q_ref[lo:hi]
        k = kv_ref[slot, hf, 0, lo4:lo4 + HG].astype(jnp.bfloat16)
        v = kv_ref[slot, hf, 1, lo4:lo4 + HG].astype(jnp.bfloat16)
        s = lax.dot_general(q, k, (((2,), (2,)), ((0,), (0,))),
                            preferred_element_type=jnp.float32)
        s = s * sc_ref[slot, 0, lo:hi]
        p = jnp.exp(s)
        den_ref[lo:hi] += jnp.sum(p, axis=2, keepdims=True)
        p = (p * sc_ref[slot, 1, lo:hi]).astype(jnp.bfloat16)
        acc_ref[lo:hi] += lax.dot_general(p, v, (((2,), (1,)), ((0,), (0,))),
                                          preferred_element_type=jnp.float32)

    def mk(src, dst, sem_idx, dev):
        return pltpu.make_async_remote_copy(
            src_ref=src, dst_ref=dst,
            send_sem=send_sems.at[sem_idx], recv_sem=recv_sems.at[sem_idx],
            device_id=(dev,), device_id_type=pl.DeviceIdType.MESH,
        )

    r1r = [mk(kv_ref.at[0, h], kv_ref.at[1, h], 0 + h, right) for h in (0, 1)]
    r1l = [mk(kv_ref.at[0, h], kv_ref.at[2, h], 2 + h, left) for h in (0, 1)]
    r2r = [mk(kv_ref.at[1, h, 0], kv_ref.at[3, h, 0], 4 + h, right)
           for h in (0, 1)]
    r2l = [mk(kv_ref.at[2, h, 1], kv_ref.at[3, h, 1], 6 + h, left)
           for h in (0, 1)]
    s1r = mk(sc_ref.at[0], sc_ref.at[1], 8, right)
    s1l = mk(sc_ref.at[0], sc_ref.at[2], 9, left)
    s2r = mk(sc_ref.at[1, 0], sc_ref.at[3, 0], 10, right)
    s2l = mk(sc_ref.at[2, 1], sc_ref.at[3, 1], 11, left)

    s1r.start()
    s1l.start()
    for r in (*r1r, *r1l):
        r.start()

    for g in range(4):
        accum(0, g)

    s1r.wait_recv()
    s2r.start()
    s1l.wait_recv()
    s2l.start()

    r1r[0].wait_recv()
    r2r[0].start()
    r1l[0].wait_recv()
    r2l[0].start()
    accum(1, 0); accum(1, 1)
    accum(2, 0); accum(2, 1)

    r1r[1].wait_recv()
    r2r[1].start()
    r1l[1].wait_recv()
    r2l[1].start()
    accum(1, 2); accum(1, 3)
    accum(2, 2); accum(2, 3)

    s2r.wait_recv()
    s2l.wait_recv()
    r2r[0].wait_recv()
    r2l[0].wait_recv()
    accum(3, 0); accum(3, 1)
    r2r[1].wait_recv()
    r2l[1].wait_recv()
    accum(3, 2); accum(3, 3)

    for r in (*r1r, *r1l, *r2r, *r2l, s1r, s1l, s2r, s2l):
        r.wait_send()

    out_ref[...] = acc_ref[...] / den_ref[...]


def kernel(Q, K, V):
    qt = (jnp.transpose(Q, (0, 2, 1, 3)) * SCALE).astype(jnp.bfloat16)
    qt = qt.reshape(BH, S_SHARD, D)
    kt = jnp.transpose(K, (0, 2, 1, 3)).reshape(BH, S_SHARD, D)
    vt = jnp.transpose(V, (0, 2, 1, 3)).reshape(BH, S_SHARD, D)

    def quant(x):
        m = jnp.max(jnp.abs(x), axis=2, keepdims=True)
        xq = jnp.round(x * (127.0 / m)).astype(jnp.int8)
        sc = jnp.transpose(m / 127.0, (0, 2, 1))
        return xq, sc

    kq, ksc = quant(kt)
    vq, vsc = quant(vt)

    out = pl.pallas_call(
        _attn_body,
        out_shape=jax.ShapeDtypeStruct((BH, S_SHARD, D), jnp.float32),
        in_specs=[pl.BlockSpec(memory_space=pltpu.VMEM)] * 5,
        out_specs=pl.BlockSpec(memory_space=pltpu.VMEM),
        scratch_shapes=[
            pltpu.VMEM((N_DEV, 2, 2, HH, S_SHARD, D), jnp.int8),
            pltpu.VMEM((N_DEV, 2, BH, 1, S_SHARD), jnp.float32),
            pltpu.VMEM((BH, S_SHARD, D), jnp.float32),
            pltpu.VMEM((BH, S_SHARD, 1), jnp.float32),
            pltpu.SemaphoreType.DMA((12,)),
            pltpu.SemaphoreType.DMA((12,)),
        ],
        compiler_params=pltpu.CompilerParams(collective_id=0),
    )(qt, kq, vq, ksc, vsc)

    return jnp.transpose(out.reshape(B, H, S_SHARD, D), (0, 2, 1, 3))


# device time: 37887 ns/iter; 4.2860x vs baseline; 1.5028x over previous
import jax
import jax.numpy as jnp
from jax import lax
from jax.experimental import pallas as pl
from jax.experimental.pallas import tpu as pltpu

N_DEV = 4
B, S_SHARD, H, D = 2, 512, 8, 64
BH = B * H
HG = 4
HH = BH // 2
SCALE = D ** -0.5


def _attn_body(q_ref, k_ref, v_ref, out_ref,
               kv_ref, sc_ref, acc_ref, den_ref, send_sems, recv_sems):
    my = lax.axis_index("i")
    left = lax.rem(my + N_DEV - 1, N_DEV)
    right = lax.rem(my + 1, N_DEV)

    acc_ref[...] = jnp.zeros_like(acc_ref)
    den_ref[...] = jnp.zeros_like(den_ref)

    def quant_store(x_ref, kvi):
        x = x_ref[...].astype(jnp.float32)
        m = jnp.maximum(jnp.max(jnp.abs(x), axis=1, keepdims=True), 1e-20)
        xq = jnp.round(x * (127.0 / m)).astype(jnp.int8)
        kv_ref[0, 0, kvi] = xq[:HH]
        kv_ref[0, 1, kvi] = xq[HH:]
        sc_ref[0, kvi] = m * (1.0 / 127.0)

    quant_store(k_ref, 0)
    quant_store(v_ref, 1)

    barrier_sem = pltpu.get_barrier_semaphore()
    for nbr in (left, right):
        pl.semaphore_signal(barrier_sem, inc=1, device_id=(nbr,),
                            device_id_type=pl.DeviceIdType.MESH)
    pl.semaphore_wait(barrier_sem, 2)

    def accum(slot, g):
        hf, l4 = divmod(g, 2)
        lo4 = l4 * HG
        lo, hi = g * HG, (g + 1) * HG
        q = q_ref[lo:hi]
        k = kv_ref[slot, hf, 0, lo4:lo4 + HG].astype(jnp.bfloat16)
        v = kv_ref[slot, hf, 1, lo4:lo4 + HG].astype(jnp.bfloat16)
        s = lax.dot_general(q, k, (((2,), (1,)), ((0,), (0,))),
                            preferred_element_type=jnp.float32)
        s = s * sc_ref[slot, 0, lo:hi]
        p = jnp.exp(s)
        den_ref[lo:hi] += jnp.sum(p, axis=2, keepdims=True)
        p = (p * sc_ref[slot, 1, lo:hi]).astype(jnp.bfloat16)
        acc_ref[lo:hi] += lax.dot_general(p, v, (((2,), (2,)), ((0,), (0,))),
                                          preferred_element_type=jnp.float32)

    def mk(src, dst, sem_idx, dev):
        return pltpu.make_async_remote_copy(
            src_ref=src, dst_ref=dst,
            send_sem=send_sems.at[sem_idx], recv_sem=recv_sems.at[sem_idx],
            device_id=(dev,), device_id_type=pl.DeviceIdType.MESH,
        )

    r1r = [mk(kv_ref.at[0, h], kv_ref.at[1, h], 0 + h, right) for h in (0, 1)]
    r1l = [mk(kv_ref.at[0, h], kv_ref.at[2, h], 2 + h, left) for h in (0, 1)]
    r2r = [mk(kv_ref.at[1, h, 0], kv_ref.at[3, h, 0], 4 + h, right)
           for h in (0, 1)]
    r2l = [mk(kv_ref.at[2, h, 1], kv_ref.at[3, h, 1], 6 + h, left)
           for h in (0, 1)]
    s1r = mk(sc_ref.at[0], sc_ref.at[1], 8, right)
    s1l = mk(sc_ref.at[0], sc_ref.at[2], 9, left)
    s2r = mk(sc_ref.at[1, 0], sc_ref.at[3, 0], 10, right)
    s2l = mk(sc_ref.at[2, 1], sc_ref.at[3, 1], 11, left)

    s1r.start()
    s1l.start()
    for r in (*r1r, *r1l):
        r.start()

    for g in range(4):
        accum(0, g)

    s1r.wait_recv()
    s2r.start()
    s1l.wait_recv()
    s2l.start()

    r1r[0].wait_recv()
    r2r[0].start()
    r1l[0].wait_recv()
    r2l[0].start()
    accum(1, 0); accum(1, 1)
    accum(2, 0); accum(2, 1)

    r1r[1].wait_recv()
    r2r[1].start()
    r1l[1].wait_recv()
    r2l[1].start()
    accum(1, 2); accum(1, 3)
    accum(2, 2); accum(2, 3)

    s2r.wait_recv()
    s2l.wait_recv()
    r2r[0].wait_recv()
    r2l[0].wait_recv()
    accum(3, 0); accum(3, 1)
    r2r[1].wait_recv()
    r2l[1].wait_recv()
    accum(3, 2); accum(3, 3)

    for r in (*r1r, *r1l, *r2r, *r2l, s1r, s1l, s2r, s2l):
        r.wait_send()

    out_ref[...] = acc_ref[...] / den_ref[...]


def kernel(Q, K, V):
    qt = (jnp.transpose(Q, (0, 2, 1, 3)) * SCALE).astype(jnp.bfloat16)
    qt = qt.reshape(BH, S_SHARD, D)
    kt = jnp.transpose(K, (0, 2, 3, 1)).astype(jnp.bfloat16).reshape(BH, D, S_SHARD)
    vt = jnp.transpose(V, (0, 2, 3, 1)).astype(jnp.bfloat16).reshape(BH, D, S_SHARD)

    out = pl.pallas_call(
        _attn_body,
        out_shape=jax.ShapeDtypeStruct((BH, S_SHARD, D), jnp.float32),
        in_specs=[pl.BlockSpec(memory_space=pltpu.VMEM)] * 3,
        out_specs=pl.BlockSpec(memory_space=pltpu.VMEM),
        scratch_shapes=[
            pltpu.VMEM((N_DEV, 2, 2, HH, D, S_SHARD), jnp.int8),
            pltpu.VMEM((N_DEV, 2, BH, 1, S_SHARD), jnp.float32),
            pltpu.VMEM((BH, S_SHARD, D), jnp.float32),
            pltpu.VMEM((BH, S_SHARD, 1), jnp.float32),
            pltpu.SemaphoreType.DMA((12,)),
            pltpu.SemaphoreType.DMA((12,)),
        ],
        compiler_params=pltpu.CompilerParams(collective_id=0),
    )(qt, kt, vt)

    return jnp.transpose(out.reshape(B, H, S_SHARD, D), (0, 2, 1, 3))
